# baseline (speedup 1.0000x reference)
import functools

import jax
import jax.numpy as jnp
from jax import lax
from jax.experimental import pallas as pl
from jax.experimental.pallas import tpu as pltpu
from jax.experimental.pallas import tpu_sc as plsc

N = 10000
E = 320000
HID = 128
EPS = 1e-5

NC = 2
NS = 16
HALF = N // 2
W = 144

B0 = 128
NB0 = 160
E_PAD = NS * NB0 * B0

BATCH = 64
NBUF = 4
LOOKAHEAD = 2
QUARTER = HALF // 2
QOFF = 2560
LCAP = 7168
DUMMY = 5100
H_AGG = 5120
ZROWS = H_AGG // NS

_sc_mesh = plsc.VectorSubcoreMesh(core_axis_name="c", subcore_axis_name="s")


@functools.partial(
    pl.kernel,
    out_type=(jax.ShapeDtypeStruct((NC, NS, 2, 2 * LCAP), jnp.int32),
              jax.ShapeDtypeStruct((NC, NS, 16), jnp.int32)),
    mesh=_sc_mesh,
    scratch_types=[
        pltpu.VMEM((NB0, B0), jnp.int32),
        pltpu.VMEM((NB0, B0), jnp.int32),
        pltpu.VMEM((2 * LCAP,), jnp.int32),
        pltpu.VMEM((2 * LCAP,), jnp.int32),
        pltpu.VMEM((16,), jnp.int32),
    ],
    compiler_params=pltpu.CompilerParams(use_tc_tiling_on_sc=False,
                                         needs_layout_passes=False),
)
def _sc_compact(edges, edc, nbh, src_in, dst_in, src_c, dst_c, nb_v):
    c = lax.axis_index("c")
    s = lax.axis_index("s")
    base = c * HALF
    pltpu.sync_copy(edges.at[0, s], src_in)
    pltpu.sync_copy(edges.at[1, s], dst_in)

    def row(j, offs):
        off0, off1 = offs
        for g in range(B0 // 16):
            s16 = src_in[j, pl.ds(g * 16, 16)]
            d16 = dst_in[j, pl.ds(g * 16, 16)]
            dloc = d16 - base
            keep = (dloc >= 0) & (dloc < HALF)
            q1 = dloc >= QUARTER
            k0 = keep & jnp.logical_not(q1)
            k1 = keep & q1
            ki0 = k0.astype(jnp.int32)
            ki1 = k1.astype(jnp.int32)
            drow = dloc + q1.astype(jnp.int32) * (QOFF - QUARTER)
            pos0 = off0 + plsc.cumsum(ki0) - ki0
            plsc.store_scatter(src_c, [pos0], s16, mask=k0)
            plsc.store_scatter(dst_c, [pos0], drow, mask=k0)
            pos1 = LCAP + off1 + plsc.cumsum(ki1) - ki1
            plsc.store_scatter(src_c, [pos1], s16, mask=k1)
            plsc.store_scatter(dst_c, [pos1], drow, mask=k1)
            off0 = off0 + jnp.sum(ki0)
            off1 = off1 + jnp.sum(ki1)
        return off0, off1

    off0, off1 = lax.fori_loop(0, NB0, row, (0, 0))
    zs = jnp.zeros((16,), jnp.int32)
    dmy = jnp.full((16,), DUMMY, jnp.int32)
    for t in range(16 * BATCH // 16):
        src_c[pl.ds(off0 + t * 16, 16)] = zs
        dst_c[pl.ds(off0 + t * 16, 16)] = dmy
        src_c[pl.ds(LCAP + off1 + t * 16, 16)] = zs
        dst_c[pl.ds(LCAP + off1 + t * 16, 16)] = dmy
    nbq = jnp.maximum((off0 + BATCH - 1) // BATCH,
                      (off1 + BATCH - 1) // BATCH)
    nbq = jnp.maximum((nbq + 1) // 2 * 2, 2)
    nb_v[...] = jnp.full((16,), nbq, jnp.int32)
    pltpu.sync_copy(src_c, edc.at[c, s, 0])
    pltpu.sync_copy(dst_c, edc.at[c, s, 1])
    pltpu.sync_copy(nb_v, nbh.at[c, s])


@functools.partial(
    pl.kernel,
    out_type=jax.ShapeDtypeStruct((NC, H_AGG, W), jnp.float32),
    mesh=_sc_mesh,
    scratch_types=[
        pltpu.VMEM((2 * LCAP,), jnp.int32),
        pltpu.VMEM((2 * LCAP,), jnp.int32),
        pltpu.VMEM((16,), jnp.int32),
        [pltpu.VMEM((BATCH, W), jnp.float32) for _ in range(NBUF)],
        [pltpu.SemaphoreType.DMA for _ in range(NBUF)],
        [pltpu.SemaphoreType.DMA for _ in range(NBUF)],
        pltpu.VMEM_SHARED((H_AGG, W), jnp.float32),
    ],
    compiler_params=pltpu.CompilerParams(use_tc_tiling_on_sc=False,
                                         needs_layout_passes=False),
)
def _sc_agg(table, edc, nbh, zeros_h, out, src_v, dst_v, nb_v, rows, gsem,
            ssem, shared):
    c = lax.axis_index("c")
    s = lax.axis_index("s")
    pltpu.sync_copy(edc.at[c, s, 0], src_v)
    pltpu.sync_copy(edc.at[c, s, 1], dst_v)
    pltpu.sync_copy(nbh.at[c, s], nb_v)
    nbq = jnp.max(nb_v[...])
    nb = 2 * nbq

    def _off(i, b):
        return (b % 2) * LCAP + ((i - b) // 2 + b // 2) * BATCH

    def _gather(i, b):
        return pltpu.make_async_copy(
            table.at[src_v.at[pl.ds(_off(i, b), BATCH)]], rows[b], gsem[b])

    def _scatter(i, b):
        return pltpu.make_async_copy(
            rows[b], shared.at[dst_v.at[pl.ds(_off(i, b), BATCH)]], ssem[b])

    for b in range(LOOKAHEAD):
        _gather(b, b).start()
    pltpu.sync_copy(zeros_h, shared.at[pl.ds(s * ZROWS, ZROWS)])
    plsc.subcore_barrier()

    def group(j, carry):
        for b in range(NBUF):
            i = j * NBUF + b
            _gather(i, b).wait()

            @pl.when(i >= LOOKAHEAD)
            def _retire():
                _scatter(i - LOOKAHEAD, (b - LOOKAHEAD) % NBUF).wait()

            pltpu.async_copy(rows[b],
                             shared.at[dst_v.at[pl.ds(_off(i, b), BATCH)]],
                             ssem[b], add=True)
            f = i + LOOKAHEAD

            @pl.when(f < nb)
            def _prefetch():
                _gather(f, (b + LOOKAHEAD) % NBUF).start()
        return carry

    lax.fori_loop(0, nb // NBUF, group, 0)
    _scatter(nb - 2, NBUF - 2).wait()
    _scatter(nb - 1, NBUF - 1).wait()
    plsc.subcore_barrier()
    pltpu.sync_copy(shared.at[pl.ds(s * ZROWS, ZROWS)],
                    out.at[c, pl.ds(s * ZROWS, ZROWS)])


def _sage_bn_relu(parts, h, Wl, bl, Wr, g, b):
    full = jnp.concatenate(
        [parts[0, :QUARTER, :], parts[0, QOFF:QOFF + QUARTER, :],
         parts[1, :QUARTER, :], parts[1, QOFF:QOFF + QUARTER, :]], axis=0)
    cnt = jnp.maximum(full[:, HID:HID + 1], 1.0)
    mean = full[:, :HID] / cnt
    z = (jnp.dot(mean, Wl, preferred_element_type=jnp.float32)
         + bl[None, :]
         + jnp.dot(h[:, :HID], Wr, preferred_element_type=jnp.float32))
    mu = jnp.mean(z, axis=0, keepdims=True)
    var = jnp.mean((z - mu) * (z - mu), axis=0, keepdims=True)
    y = (z - mu) * lax.rsqrt(var + EPS) * g[None, :] + b[None, :]
    return jnp.maximum(y, 0.0)


def _tc_layer_body(parts, h, Wl, bl, Wr, g, b, out):
    y = _sage_bn_relu(parts[...], h[...], Wl[...], bl[...], Wr[...],
                      g[...], b[...])
    out[:, :HID] = y
    lane = lax.broadcasted_iota(jnp.int32, (N, W - HID), 1)
    out[:, HID:] = jnp.where(lane == 0, 1.0, 0.0)


def _tc_last_body(parts, h, Wl, bl, Wr, g, b, wlin, blin, out):
    y = _sage_bn_relu(parts[...], h[...], Wl[...], bl[...], Wr[...],
                      g[...], b[...])
    out[...] = (jnp.dot(y, wlin[...], preferred_element_type=jnp.float32)
                + blin[0])


_tc_layer = pl.pallas_call(
    _tc_layer_body,
    out_shape=jax.ShapeDtypeStruct((N, W), jnp.float32),
)

_tc_last = pl.pallas_call(
    _tc_last_body,
    out_shape=jax.ShapeDtypeStruct((N, 1), jnp.float32),
)


def kernel(x, edge_index, Wl0, bl0, Wr0, gamma0, beta0, Wl1, bl1, Wr1,
           gamma1, beta1, Wl2, bl2, Wr2, gamma2, beta2, Wlin, blin):
    pad = E_PAD - E
    src = jnp.concatenate([edge_index[0], jnp.zeros((pad,), jnp.int32)])
    dst = jnp.concatenate([edge_index[1], jnp.full((pad,), N, jnp.int32)])
    edges = jnp.stack([src, dst]).reshape(2, NS, NB0, B0)
    zeros_h = jnp.zeros((ZROWS, W), jnp.float32)
    h = jnp.concatenate(
        [x, jnp.ones((N, 1), jnp.float32),
         jnp.zeros((N, W - HID - 1), jnp.float32)], axis=1)

    edc, nbh = _sc_compact(edges)
    parts = _sc_agg(h, edc, nbh, zeros_h)
    h = _tc_layer(parts, h, Wl0, bl0, Wr0, gamma0, beta0)
    parts = _sc_agg(h, edc, nbh, zeros_h)
    h = _tc_layer(parts, h, Wl1, bl1, Wr1, gamma1, beta1)
    parts = _sc_agg(h, edc, nbh, zeros_h)
    out = _tc_last(parts, h, Wl2, bl2, Wr2, gamma2, beta2, Wlin, blin)
    return out[:, 0]


# speedup vs baseline: 1.1813x; 1.1813x over previous
import functools

import jax
import jax.numpy as jnp
from jax import lax
from jax.experimental import pallas as pl
from jax.experimental.pallas import tpu as pltpu
from jax.experimental.pallas import tpu_sc as plsc

N = 10000
E = 320000
HID = 128
EPS = 1e-5

NC = 2
NS = 16
HALF = N // 2
W = 144

B0 = 128
NB0 = 160
E_PAD = NS * NB0 * B0

BATCH = 128
NBUF = 2
LOOKAHEAD = 1
LCAP = 11776
DUMMY = HALF
H_AGG = 5120
ZROWS = H_AGG // NS

_sc_mesh = plsc.VectorSubcoreMesh(core_axis_name="c", subcore_axis_name="s")


@functools.partial(
    pl.kernel,
    out_type=(jax.ShapeDtypeStruct((NC, NS, 2, LCAP), jnp.int32),
              jax.ShapeDtypeStruct((NC, NS, 16), jnp.int32)),
    mesh=_sc_mesh,
    scratch_types=[
        pltpu.VMEM((NB0, B0), jnp.int32),
        pltpu.VMEM((NB0, B0), jnp.int32),
        pltpu.VMEM((LCAP,), jnp.int32),
        pltpu.VMEM((LCAP,), jnp.int32),
        pltpu.VMEM((16,), jnp.int32),
    ],
    compiler_params=pltpu.CompilerParams(use_tc_tiling_on_sc=False,
                                         needs_layout_passes=False),
)
def _sc_compact(edges, edc, nbh, src_in, dst_in, src_c, dst_c, nb_v):
    c = lax.axis_index("c")
    s = lax.axis_index("s")
    base = c * HALF
    pltpu.sync_copy(edges.at[0, s], src_in)
    pltpu.sync_copy(edges.at[1, s], dst_in)

    def row(j, off):
        for g in range(B0 // 16):
            s16 = src_in[j, pl.ds(g * 16, 16)]
            d16 = dst_in[j, pl.ds(g * 16, 16)]
            keep = (d16 >= base) & (d16 < base + HALF)
            ks = keep.astype(jnp.int32)
            pos = off + plsc.cumsum(ks) - ks
            plsc.store_scatter(src_c, [pos], s16, mask=keep)
            plsc.store_scatter(dst_c, [pos], d16 - base, mask=keep)
            off = off + jnp.sum(ks)
        return off

    off = lax.fori_loop(0, NB0, row, 0)
    zs = jnp.zeros((16,), jnp.int32)
    dmy = jnp.full((16,), DUMMY, jnp.int32)
    for t in range(NBUF * BATCH // 16):
        src_c[pl.ds(off + t * 16, 16)] = zs
        dst_c[pl.ds(off + t * 16, 16)] = dmy
    nb = (off + BATCH - 1) // BATCH
    nb = (nb + NBUF - 1) // NBUF * NBUF
    nb_v[...] = jnp.full((16,), nb, jnp.int32)
    pltpu.sync_copy(src_c, edc.at[c, s, 0])
    pltpu.sync_copy(dst_c, edc.at[c, s, 1])
    pltpu.sync_copy(nb_v, nbh.at[c, s])


@functools.partial(
    pl.kernel,
    out_type=jax.ShapeDtypeStruct((NC, H_AGG, W), jnp.float32),
    mesh=_sc_mesh,
    scratch_types=[
        pltpu.VMEM((LCAP,), jnp.int32),
        pltpu.VMEM((LCAP,), jnp.int32),
        pltpu.VMEM((16,), jnp.int32),
        [pltpu.VMEM((BATCH, W), jnp.float32) for _ in range(NBUF)],
        [pltpu.SemaphoreType.DMA for _ in range(NBUF)],
        [pltpu.SemaphoreType.DMA for _ in range(NBUF)],
        pltpu.VMEM_SHARED((H_AGG, W), jnp.float32),
    ],
    compiler_params=pltpu.CompilerParams(use_tc_tiling_on_sc=False,
                                         needs_layout_passes=False),
)
def _sc_agg(table, edc, nbh, zeros_h, out, src_v, dst_v, nb_v, rows, gsem,
            ssem, shared):
    c = lax.axis_index("c")
    s = lax.axis_index("s")
    pltpu.sync_copy(edc.at[c, s, 0], src_v)
    pltpu.sync_copy(edc.at[c, s, 1], dst_v)
    pltpu.sync_copy(nbh.at[c, s], nb_v)
    nb = jnp.maximum(jnp.max(nb_v[...]), NBUF)

    def _gather(i, b):
        return pltpu.make_async_copy(
            table.at[src_v.at[pl.ds(i * BATCH, BATCH)]], rows[b], gsem[b])

    def _scatter(i, b):
        return pltpu.make_async_copy(
            rows[b], shared.at[dst_v.at[pl.ds(i * BATCH, BATCH)]], ssem[b])

    for b in range(LOOKAHEAD):
        _gather(b, b).start()
    pltpu.sync_copy(zeros_h, shared.at[pl.ds(s * ZROWS, ZROWS)])
    plsc.subcore_barrier()

    def group(j, carry):
        for b in range(NBUF):
            i = j * NBUF + b
            _gather(i, b).wait()

            @pl.when(i >= 1)
            def _retire_prev():
                _scatter(i - 1, (b - 1) % NBUF).wait()

            pltpu.async_copy(rows[b],
                             shared.at[dst_v.at[pl.ds(i * BATCH, BATCH)]],
                             ssem[b], add=True)
            f = i + LOOKAHEAD

            @pl.when(f < nb)
            def _prefetch():
                _gather(f, (b + LOOKAHEAD) % NBUF).start()
        return carry

    lax.fori_loop(0, nb // NBUF, group, 0)
    _scatter(nb - 1, NBUF - 1).wait()
    plsc.subcore_barrier()
    pltpu.sync_copy(shared.at[pl.ds(s * ZROWS, ZROWS)],
                    out.at[c, pl.ds(s * ZROWS, ZROWS)])


def _sage_bn_relu(parts, h, Wl, bl, Wr, g, b):
    full = jnp.concatenate([parts[0, :HALF, :], parts[1, :HALF, :]], axis=0)
    cnt = jnp.maximum(full[:, HID:HID + 1], 1.0)
    mean = full[:, :HID] / cnt
    z = (jnp.dot(mean, Wl, preferred_element_type=jnp.float32)
         + bl[None, :]
         + jnp.dot(h[:, :HID], Wr, preferred_element_type=jnp.float32))
    mu = jnp.mean(z, axis=0, keepdims=True)
    var = jnp.mean((z - mu) * (z - mu), axis=0, keepdims=True)
    y = (z - mu) * lax.rsqrt(var + EPS) * g[None, :] + b[None, :]
    return jnp.maximum(y, 0.0)


def _tc_layer_body(parts, h, Wl, bl, Wr, g, b, out):
    y = _sage_bn_relu(parts[...], h[...], Wl[...], bl[...], Wr[...],
                      g[...], b[...])
    out[:, :HID] = y
    lane = lax.broadcasted_iota(jnp.int32, (N, W - HID), 1)
    out[:, HID:] = jnp.where(lane == 0, 1.0, 0.0)


def _tc_last_body(parts, h, Wl, bl, Wr, g, b, wlin, blin, out):
    y = _sage_bn_relu(parts[...], h[...], Wl[...], bl[...], Wr[...],
                      g[...], b[...])
    out[...] = (jnp.dot(y, wlin[...], preferred_element_type=jnp.float32)
                + blin[0])


_tc_layer = pl.pallas_call(
    _tc_layer_body,
    out_shape=jax.ShapeDtypeStruct((N, W), jnp.float32),
)

_tc_last = pl.pallas_call(
    _tc_last_body,
    out_shape=jax.ShapeDtypeStruct((N, 1), jnp.float32),
)


def kernel(x, edge_index, Wl0, bl0, Wr0, gamma0, beta0, Wl1, bl1, Wr1,
           gamma1, beta1, Wl2, bl2, Wr2, gamma2, beta2, Wlin, blin):
    pad = E_PAD - E
    src = jnp.concatenate([edge_index[0], jnp.zeros((pad,), jnp.int32)])
    dst = jnp.concatenate([edge_index[1], jnp.full((pad,), N, jnp.int32)])
    edges = jnp.stack([src, dst]).reshape(2, NS, NB0, B0)
    zeros_h = jnp.zeros((ZROWS, W), jnp.float32)
    h = jnp.concatenate(
        [x, jnp.ones((N, 1), jnp.float32),
         jnp.zeros((N, W - HID - 1), jnp.float32)], axis=1)

    edc, nbh = _sc_compact(edges)
    parts = _sc_agg(h, edc, nbh, zeros_h)
    h = _tc_layer(parts, h, Wl0, bl0, Wr0, gamma0, beta0)
    parts = _sc_agg(h, edc, nbh, zeros_h)
    h = _tc_layer(parts, h, Wl1, bl1, Wr1, gamma1, beta1)
    parts = _sc_agg(h, edc, nbh, zeros_h)
    out = _tc_last(parts, h, Wl2, bl2, Wr2, gamma2, beta2, Wlin, blin)
    return out[:, 0]


# speedup vs baseline: 1.3097x; 1.1087x over previous
import functools

import jax
import jax.numpy as jnp
from jax import lax
from jax.experimental import pallas as pl
from jax.experimental.pallas import tpu as pltpu
from jax.experimental.pallas import tpu_sc as plsc

N = 10000
E = 320000
HID = 128
EPS = 1e-5

NC = 2
NS = 16
HALF = N // 2
W = 144

B0 = 128
NB0 = 160
E_PAD = NS * NB0 * B0

BATCH = 64
NBUF = 4
LOOKAHEAD = 3
LCAP = 11776
DUMMY = HALF
H_AGG = 5120
ZROWS = H_AGG // NS

_sc_mesh = plsc.VectorSubcoreMesh(core_axis_name="c", subcore_axis_name="s")


@functools.partial(
    pl.kernel,
    out_type=(jax.ShapeDtypeStruct((NC, NS, 2, LCAP), jnp.int32),
              jax.ShapeDtypeStruct((NC, NS, 16), jnp.int32)),
    mesh=_sc_mesh,
    scratch_types=[
        pltpu.VMEM((NB0, B0), jnp.int32),
        pltpu.VMEM((NB0, B0), jnp.int32),
        pltpu.VMEM((LCAP,), jnp.int32),
        pltpu.VMEM((LCAP,), jnp.int32),
        pltpu.VMEM((16,), jnp.int32),
    ],
    compiler_params=pltpu.CompilerParams(use_tc_tiling_on_sc=False,
                                         needs_layout_passes=False),
)
def _sc_compact(edges, edc, nbh, src_in, dst_in, src_c, dst_c, nb_v):
    c = lax.axis_index("c")
    s = lax.axis_index("s")
    base = c * HALF
    pltpu.sync_copy(edges.at[0, s], src_in)
    pltpu.sync_copy(edges.at[1, s], dst_in)

    def row(j, off):
        for g in range(B0 // 16):
            s16 = src_in[j, pl.ds(g * 16, 16)]
            d16 = dst_in[j, pl.ds(g * 16, 16)]
            keep = (d16 >= base) & (d16 < base + HALF)
            ks = keep.astype(jnp.int32)
            pos = off + plsc.cumsum(ks) - ks
            plsc.store_scatter(src_c, [pos], s16, mask=keep)
            plsc.store_scatter(dst_c, [pos], d16 - base, mask=keep)
            off = off + jnp.sum(ks)
        return off

    off = lax.fori_loop(0, NB0, row, 0)
    zs = jnp.zeros((16,), jnp.int32)
    dmy = jnp.full((16,), DUMMY, jnp.int32)
    for t in range(NBUF * BATCH // 16):
        src_c[pl.ds(off + t * 16, 16)] = zs
        dst_c[pl.ds(off + t * 16, 16)] = dmy
    nb = (off + BATCH - 1) // BATCH
    nb = (nb + NBUF - 1) // NBUF * NBUF
    nb_v[...] = jnp.full((16,), nb, jnp.int32)
    pltpu.sync_copy(src_c, edc.at[c, s, 0])
    pltpu.sync_copy(dst_c, edc.at[c, s, 1])
    pltpu.sync_copy(nb_v, nbh.at[c, s])


@functools.partial(
    pl.kernel,
    out_type=jax.ShapeDtypeStruct((NC, H_AGG, W), jnp.float32),
    mesh=_sc_mesh,
    scratch_types=[
        pltpu.VMEM((LCAP,), jnp.int32),
        pltpu.VMEM((LCAP,), jnp.int32),
        pltpu.VMEM((16,), jnp.int32),
        [pltpu.VMEM((BATCH, W), jnp.float32) for _ in range(NBUF)],
        [pltpu.SemaphoreType.DMA for _ in range(NBUF)],
        [pltpu.SemaphoreType.DMA for _ in range(NBUF)],
        pltpu.VMEM_SHARED((H_AGG, W), jnp.float32),
    ],
    compiler_params=pltpu.CompilerParams(use_tc_tiling_on_sc=False,
                                         needs_layout_passes=False),
)
def _sc_agg(table, edc, nbh, zeros_h, out, src_v, dst_v, nb_v, rows, gsem,
            ssem, shared):
    c = lax.axis_index("c")
    s = lax.axis_index("s")
    pltpu.sync_copy(edc.at[c, s, 0], src_v)
    pltpu.sync_copy(edc.at[c, s, 1], dst_v)
    pltpu.sync_copy(nbh.at[c, s], nb_v)
    nb = jnp.maximum(jnp.max(nb_v[...]), NBUF)

    def _gather(i, b):
        return pltpu.make_async_copy(
            table.at[src_v.at[pl.ds(i * BATCH, BATCH)]], rows[b], gsem[b])

    def _scatter(i, b):
        return pltpu.make_async_copy(
            rows[b], shared.at[dst_v.at[pl.ds(i * BATCH, BATCH)]], ssem[b])

    for b in range(LOOKAHEAD):
        _gather(b, b).start()
    pltpu.sync_copy(zeros_h, shared.at[pl.ds(s * ZROWS, ZROWS)])
    plsc.subcore_barrier()

    def group(j, carry):
        for b in range(NBUF):
            i = j * NBUF + b
            _gather(i, b).wait()

            @pl.when(i >= 1)
            def _retire_prev():
                _scatter(i - 1, (b - 1) % NBUF).wait()

            pltpu.async_copy(rows[b],
                             shared.at[dst_v.at[pl.ds(i * BATCH, BATCH)]],
                             ssem[b], add=True)
            f = i + LOOKAHEAD

            @pl.when(f < nb)
            def _prefetch():
                _gather(f, (b + LOOKAHEAD) % NBUF).start()
        return carry

    lax.fori_loop(0, nb // NBUF, group, 0)
    _scatter(nb - 1, NBUF - 1).wait()
    plsc.subcore_barrier()
    pltpu.sync_copy(shared.at[pl.ds(s * ZROWS, ZROWS)],
                    out.at[c, pl.ds(s * ZROWS, ZROWS)])


def _sage_bn_relu(parts, h, Wl, bl, Wr, g, b):
    full = jnp.concatenate([parts[0, :HALF, :], parts[1, :HALF, :]], axis=0)
    cnt = jnp.maximum(full[:, HID:HID + 1], 1.0)
    mean = full[:, :HID] / cnt
    z = (jnp.dot(mean, Wl, preferred_element_type=jnp.float32)
         + bl[None, :]
         + jnp.dot(h[:, :HID], Wr, preferred_element_type=jnp.float32))
    mu = jnp.mean(z, axis=0, keepdims=True)
    var = jnp.mean((z - mu) * (z - mu), axis=0, keepdims=True)
    y = (z - mu) * lax.rsqrt(var + EPS) * g[None, :] + b[None, :]
    return jnp.maximum(y, 0.0)


def _tc_layer_body(parts, h, Wl, bl, Wr, g, b, out):
    y = _sage_bn_relu(parts[...], h[...], Wl[...], bl[...], Wr[...],
                      g[...], b[...])
    out[:, :HID] = y
    lane = lax.broadcasted_iota(jnp.int32, (N, W - HID), 1)
    out[:, HID:] = jnp.where(lane == 0, 1.0, 0.0)


def _tc_last_body(parts, h, Wl, bl, Wr, g, b, wlin, blin, out):
    y = _sage_bn_relu(parts[...], h[...], Wl[...], bl[...], Wr[...],
                      g[...], b[...])
    out[...] = (jnp.dot(y, wlin[...], preferred_element_type=jnp.float32)
                + blin[0])


_tc_layer = pl.pallas_call(
    _tc_layer_body,
    out_shape=jax.ShapeDtypeStruct((N, W), jnp.float32),
)

_tc_last = pl.pallas_call(
    _tc_last_body,
    out_shape=jax.ShapeDtypeStruct((N, 1), jnp.float32),
)


def kernel(x, edge_index, Wl0, bl0, Wr0, gamma0, beta0, Wl1, bl1, Wr1,
           gamma1, beta1, Wl2, bl2, Wr2, gamma2, beta2, Wlin, blin):
    pad = E_PAD - E
    src = jnp.concatenate([edge_index[0], jnp.zeros((pad,), jnp.int32)])
    dst = jnp.concatenate([edge_index[1], jnp.full((pad,), N, jnp.int32)])
    edges = jnp.stack([src, dst]).reshape(2, NS, NB0, B0)
    zeros_h = jnp.zeros((ZROWS, W), jnp.float32)
    h = jnp.concatenate(
        [x, jnp.ones((N, 1), jnp.float32),
         jnp.zeros((N, W - HID - 1), jnp.float32)], axis=1)

    edc, nbh = _sc_compact(edges)
    parts = _sc_agg(h, edc, nbh, zeros_h)
    h = _tc_layer(parts, h, Wl0, bl0, Wr0, gamma0, beta0)
    parts = _sc_agg(h, edc, nbh, zeros_h)
    h = _tc_layer(parts, h, Wl1, bl1, Wr1, gamma1, beta1)
    parts = _sc_agg(h, edc, nbh, zeros_h)
    out = _tc_last(parts, h, Wl2, bl2, Wr2, gamma2, beta2, Wlin, blin)
    return out[:, 0]
